# baseline (speedup 1.0000x reference)
import functools
import math

import jax
import jax.numpy as jnp
import numpy as np
from jax import lax
from jax.experimental import pallas as pl
from jax.experimental.pallas import tpu as pltpu
from jax.experimental.pallas import tpu_sc as plsc

H_B, W_B = 512, 512
H_R, W_R = 64, 2048
Z_MIN, Z_MAX = -4.0, 2.0
Z_BINS = 30
Z_LOW = -1.73
PHI_MIN, PHI_MAX = -math.pi, math.pi
THETA_MIN, THETA_MAX = math.radians(-25.0), math.radians(3.0)
XMIN, XMAX, YMIN, YMAX = -50.0, 50.0, -50.0, 50.0

_N = H_B * W_B


def _static_layout():
    y_lin = np.linspace(YMAX, YMIN, H_B).astype(np.float32)
    x_lin = np.linspace(XMIN, XMAX, W_B).astype(np.float32)
    yg, xg = np.meshgrid(y_lin, x_lin, indexing="ij")
    rho = np.sqrt(xg ** 2 + yg ** 2).ravel().astype(np.float32)
    phi = np.arctan2(yg, xg).ravel().astype(np.float32)
    col = (phi - np.float32(PHI_MIN)) / np.float32(PHI_MAX - PHI_MIN)
    col = col * np.float32(W_R - 1)
    col = np.clip(np.round(col), 0, W_R - 1).astype(np.int32)

    theta_low = np.arctan2(np.float32(Z_LOW), rho).astype(np.float32)
    row_low = (np.float32(THETA_MAX) - theta_low) / np.float32(
        THETA_MAX - THETA_MIN) * np.float32(H_R - 1)
    row_low = np.clip(np.round(row_low), 0, H_R - 1).astype(np.int32)

    order = np.argsort(col, kind="stable")
    counts = np.bincount(col, minlength=W_R)
    kmax = int(counts.max())
    K = ((kmax + 7) // 8) * 8

    idx_pad = np.zeros((K, W_R), dtype=np.int32)
    valid = np.zeros((K, W_R), dtype=bool)
    offs = np.zeros(W_R + 1, dtype=np.int64)
    np.cumsum(counts, out=offs[1:])
    for c in range(W_R):
        n = counts[c]
        if n:
            idx_pad[:n, c] = order[offs[c]:offs[c + 1]]
            valid[:n, c] = True
    rho_slot = rho[idx_pad]
    row_low_slot = row_low[idx_pad]
    return K, idx_pad, valid, rho_slot, row_low_slot


_K, _IDX_PAD, _VALID, _RHO_SLOT, _ROW_LOW_SLOT = _static_layout()

_CB = 128
_NCB = W_R // _CB


_RG = 8


def _rv_body(feat_ref, s_ref, e_ref, o_ref):
    r0 = pl.program_id(2) * _RG
    s = s_ref[0]
    e = e_ref[0]
    neg = jnp.float32(-np.inf)
    for ch in range(32):
        f = feat_ref[0, ch]
        for j in range(_RG):
            r = r0 + j
            m = (s <= r) & (r <= e)
            v = jnp.where(m, f, neg)
            mx = jnp.max(v, axis=0)
            o_ref[0, ch, j, :] = jnp.where(mx == neg, jnp.float32(0.0), mx)


def _pallas_rv(featp, startp, endp, interpret=False):
    B, C = featp.shape[0], featp.shape[1]
    grid = (B, _NCB, H_R // _RG)
    return pl.pallas_call(
        _rv_body,
        grid=grid,
        in_specs=[
            pl.BlockSpec((1, C, _K, _CB), lambda b, cb, r: (b, 0, 0, cb)),
            pl.BlockSpec((1, _K, _CB), lambda b, cb, r: (b, 0, cb)),
            pl.BlockSpec((1, _K, _CB), lambda b, cb, r: (b, 0, cb)),
        ],
        out_specs=pl.BlockSpec((1, C, _RG, _CB), lambda b, cb, r: (b, 0, r, cb)),
        out_shape=jax.ShapeDtypeStruct((B, C, H_R, W_R), jnp.float32),
        interpret=interpret,
    )(featp, startp, endp)


_NW = 32
_ROW = 128
_TOT = 2 * _K * W_R
_PER_W = _TOT // _NW
_CH = 512
_ITERS = _PER_W // _CH


def _sc_gather(src, idx):
    mesh = plsc.VectorSubcoreMesh(core_axis_name="c", subcore_axis_name="s")

    @functools.partial(
        pl.kernel,
        mesh=mesh,
        out_type=jax.ShapeDtypeStruct((_TOT, _ROW), jnp.float32),
        scratch_types=[
            pltpu.VMEM((_CH,), jnp.int32),
            pltpu.VMEM((_CH, _ROW), jnp.float32),
            pltpu.SemaphoreType.DMA,
        ],
    )
    def gk(src_hbm, idx_hbm, out_hbm, idx_v, rows_v, sem):
        wid = lax.axis_index("s") * 2 + lax.axis_index("c")
        base = wid * _PER_W

        def body(i, carry):
            off = base + i * _CH
            pltpu.sync_copy(idx_hbm.at[pl.ds(off, _CH)], idx_v)
            pltpu.async_copy(src_hbm.at[idx_v], rows_v, sem).wait()
            pltpu.sync_copy(rows_v, out_hbm.at[pl.ds(off, _CH)])
            return carry

        lax.fori_loop(0, _ITERS, body, 0)

    return gk(src, idx)


def _prep(bev_feat, bev_z_bin):
    B, C = bev_feat.shape[0], bev_feat.shape[1]
    valid = jnp.asarray(_VALID)

    dz = (Z_MAX - Z_MIN) / Z_BINS
    z_hint = bev_z_bin[:, 0].astype(jnp.float32) * dz + (Z_MIN + dz / 2.0)
    bev_t = bev_feat.reshape(B, C, -1).transpose(0, 2, 1)
    src = jnp.concatenate(
        [bev_t, z_hint.reshape(B, _N, 1),
         jnp.zeros((B, _N, _ROW - C - 1), jnp.float32)], axis=2)
    src = src.reshape(B * _N, _ROW)

    g2 = jnp.asarray(
        (_IDX_PAD.reshape(1, -1) + (np.arange(2, dtype=np.int64) * _N)[:, None])
        .reshape(-1).astype(np.int32))
    go = _sc_gather(src, g2).reshape(B, _K, W_R, _ROW)

    featp = go[..., :C].transpose(0, 3, 1, 2)

    zp = go[..., C]
    rho_slot = jnp.asarray(_RHO_SLOT)
    row_low_slot = jnp.asarray(_ROW_LOW_SLOT)
    theta_high = jnp.arctan2(zp, rho_slot[None])
    row_high = (THETA_MAX - theta_high) / (THETA_MAX - THETA_MIN) * (H_R - 1)
    row_high = jnp.clip(jnp.round(row_high), 0, H_R - 1).astype(jnp.int32)
    startp = jnp.minimum(row_low_slot[None], row_high)
    endp = jnp.maximum(row_low_slot[None], row_high)
    startp = jnp.where(valid[None], startp, jnp.int32(H_R))
    endp = jnp.where(valid[None], endp, jnp.int32(-1))
    return featp, startp, endp


def kernel(bev_feat, bev_z_bin):
    featp, startp, endp = _prep(bev_feat, bev_z_bin)
    return _pallas_rv(featp, startp, endp)


# speedup vs baseline: 1.0005x; 1.0005x over previous
import functools
import math

import jax
import jax.numpy as jnp
import numpy as np
from jax import lax
from jax.experimental import pallas as pl
from jax.experimental.pallas import tpu as pltpu
from jax.experimental.pallas import tpu_sc as plsc

H_B, W_B = 512, 512
H_R, W_R = 64, 2048
Z_MIN, Z_MAX = -4.0, 2.0
Z_BINS = 30
Z_LOW = -1.73
PHI_MIN, PHI_MAX = -math.pi, math.pi
THETA_MIN, THETA_MAX = math.radians(-25.0), math.radians(3.0)
XMIN, XMAX, YMIN, YMAX = -50.0, 50.0, -50.0, 50.0

_N = H_B * W_B


def _static_layout():
    y_lin = np.linspace(YMAX, YMIN, H_B).astype(np.float32)
    x_lin = np.linspace(XMIN, XMAX, W_B).astype(np.float32)
    yg, xg = np.meshgrid(y_lin, x_lin, indexing="ij")
    rho = np.sqrt(xg ** 2 + yg ** 2).ravel().astype(np.float32)
    phi = np.arctan2(yg, xg).ravel().astype(np.float32)
    col = (phi - np.float32(PHI_MIN)) / np.float32(PHI_MAX - PHI_MIN)
    col = col * np.float32(W_R - 1)
    col = np.clip(np.round(col), 0, W_R - 1).astype(np.int32)

    theta_low = np.arctan2(np.float32(Z_LOW), rho).astype(np.float32)
    row_low = (np.float32(THETA_MAX) - theta_low) / np.float32(
        THETA_MAX - THETA_MIN) * np.float32(H_R - 1)
    row_low = np.clip(np.round(row_low), 0, H_R - 1).astype(np.int32)

    order = np.argsort(col, kind="stable")
    counts = np.bincount(col, minlength=W_R)
    kmax = int(counts.max())
    K = ((kmax + 7) // 8) * 8

    idx_pad = np.zeros((K, W_R), dtype=np.int32)
    valid = np.zeros((K, W_R), dtype=bool)
    offs = np.zeros(W_R + 1, dtype=np.int64)
    np.cumsum(counts, out=offs[1:])
    for c in range(W_R):
        n = counts[c]
        if n:
            idx_pad[:n, c] = order[offs[c]:offs[c + 1]]
            valid[:n, c] = True
    rho_slot = rho[idx_pad]
    row_low_slot = row_low[idx_pad]
    return K, idx_pad, valid, rho_slot, row_low_slot


_K, _IDX_PAD, _VALID, _RHO_SLOT, _ROW_LOW_SLOT = _static_layout()

_CB = 128
_NCB = W_R // _CB


_RG = 8


def _rv_body(feat_ref, s_ref, e_ref, o_ref):
    r0 = pl.program_id(2) * _RG
    s = s_ref[0]
    e = e_ref[0]
    neg = jnp.float32(-np.inf)
    for ch in range(32):
        f = feat_ref[0, ch]
        for j in range(_RG):
            r = r0 + j
            m = (s <= r) & (r <= e)
            v = jnp.where(m, f, neg)
            mx = jnp.max(v, axis=0)
            o_ref[0, ch, j, :] = jnp.where(mx == neg, jnp.float32(0.0), mx)


def _pallas_rv(featp, startp, endp, interpret=False):
    B, C = featp.shape[0], featp.shape[1]
    grid = (B, _NCB, H_R // _RG)
    return pl.pallas_call(
        _rv_body,
        grid=grid,
        in_specs=[
            pl.BlockSpec((1, C, _K, _CB), lambda b, cb, r: (b, 0, 0, cb)),
            pl.BlockSpec((1, _K, _CB), lambda b, cb, r: (b, 0, cb)),
            pl.BlockSpec((1, _K, _CB), lambda b, cb, r: (b, 0, cb)),
        ],
        out_specs=pl.BlockSpec((1, C, _RG, _CB), lambda b, cb, r: (b, 0, r, cb)),
        out_shape=jax.ShapeDtypeStruct((B, C, H_R, W_R), jnp.float32),
        interpret=interpret,
    )(featp, startp, endp)


_NW = 32
_ROW = 128
_TOT = 2 * _K * W_R
_PER_W = _TOT // _NW
_GCH = 128
_NB = 4
_SS = _GCH * _NB
_NSS = _PER_W // _SS


def _sc_gather(src, idx):
    mesh = plsc.VectorSubcoreMesh(core_axis_name="c", subcore_axis_name="s")

    @functools.partial(
        pl.kernel,
        mesh=mesh,
        out_type=jax.ShapeDtypeStruct((_TOT, _ROW), jnp.float32),
        scratch_types=[
            pltpu.VMEM((_PER_W,), jnp.int32),
            pltpu.VMEM((_SS, _ROW), jnp.float32),
            pltpu.SemaphoreType.DMA,
            pltpu.SemaphoreType.DMA,
        ],
    )
    def gk(src_hbm, idx_hbm, out_hbm, idx_all, rows_v, sg, sw):
        wid = lax.axis_index("s") * 2 + lax.axis_index("c")
        base = wid * _PER_W

        def g_copies(s):
            for n in range(_NB):
                yield (src_hbm.at[idx_all.at[pl.ds(s * _SS + n * _GCH, _GCH)]],
                       rows_v.at[pl.ds(n * _GCH, _GCH)])

        def wb_args(s):
            return rows_v, out_hbm.at[pl.ds(base + s * _SS, _SS)]

        pltpu.sync_copy(idx_hbm.at[pl.ds(base, _PER_W)], idx_all)
        for a, b in g_copies(0):
            pltpu.async_copy(a, b, sg)

        def body(s, carry):
            for a, b in g_copies(s):
                pltpu.make_async_copy(a, b, sg).wait()
            pltpu.async_copy(*wb_args(s), sw)
            pltpu.make_async_copy(rows_v, out_hbm.at[pl.ds(base + s * _SS, _SS)], sw).wait()
            for a, b in g_copies(s + 1):
                pltpu.async_copy(a, b, sg)
            return carry

        lax.fori_loop(0, _NSS - 1, body, 0)

        s_last = _NSS - 1
        for a, b in g_copies(s_last):
            pltpu.make_async_copy(a, b, sg).wait()
        pltpu.async_copy(*wb_args(s_last), sw)
        pltpu.make_async_copy(rows_v, out_hbm.at[pl.ds(base + s_last * _SS, _SS)], sw).wait()

    return gk(src, idx)


def _prep(bev_feat, bev_z_bin):
    B, C = bev_feat.shape[0], bev_feat.shape[1]
    valid = jnp.asarray(_VALID)

    dz = (Z_MAX - Z_MIN) / Z_BINS
    z_hint = bev_z_bin[:, 0].astype(jnp.float32) * dz + (Z_MIN + dz / 2.0)
    bev_t = bev_feat.reshape(B, C, -1).transpose(0, 2, 1)
    src = jnp.concatenate(
        [bev_t, z_hint.reshape(B, _N, 1),
         jnp.zeros((B, _N, _ROW - C - 1), jnp.float32)], axis=2)
    src = src.reshape(B * _N, _ROW)

    g2 = jnp.asarray(
        (_IDX_PAD.reshape(1, -1) + (np.arange(2, dtype=np.int64) * _N)[:, None])
        .reshape(-1).astype(np.int32))
    go = _sc_gather(src, g2).reshape(B, _K, W_R, _ROW)

    featp = go[..., :C].transpose(0, 3, 1, 2)

    zp = go[..., C]
    rho_slot = jnp.asarray(_RHO_SLOT)
    row_low_slot = jnp.asarray(_ROW_LOW_SLOT)
    theta_high = jnp.arctan2(zp, rho_slot[None])
    row_high = (THETA_MAX - theta_high) / (THETA_MAX - THETA_MIN) * (H_R - 1)
    row_high = jnp.clip(jnp.round(row_high), 0, H_R - 1).astype(jnp.int32)
    startp = jnp.minimum(row_low_slot[None], row_high)
    endp = jnp.maximum(row_low_slot[None], row_high)
    startp = jnp.where(valid[None], startp, jnp.int32(H_R))
    endp = jnp.where(valid[None], endp, jnp.int32(-1))
    return featp, startp, endp


def kernel(bev_feat, bev_z_bin):
    featp, startp, endp = _prep(bev_feat, bev_z_bin)
    return _pallas_rv(featp, startp, endp)


# speedup vs baseline: 2.4567x; 2.4556x over previous
import functools
import math

import jax
import jax.numpy as jnp
import numpy as np
from jax import lax
from jax.experimental import pallas as pl
from jax.experimental.pallas import tpu as pltpu
from jax.experimental.pallas import tpu_sc as plsc

H_B, W_B = 512, 512
H_R, W_R = 64, 2048
Z_MIN, Z_MAX = -4.0, 2.0
Z_BINS = 30
Z_LOW = -1.73
PHI_MIN, PHI_MAX = -math.pi, math.pi
THETA_MIN, THETA_MAX = math.radians(-25.0), math.radians(3.0)
XMIN, XMAX, YMIN, YMAX = -50.0, 50.0, -50.0, 50.0

_N = H_B * W_B


def _static_layout():
    y_lin = np.linspace(YMAX, YMIN, H_B).astype(np.float32)
    x_lin = np.linspace(XMIN, XMAX, W_B).astype(np.float32)
    yg, xg = np.meshgrid(y_lin, x_lin, indexing="ij")
    rho = np.sqrt(xg ** 2 + yg ** 2).ravel().astype(np.float32)
    phi = np.arctan2(yg, xg).ravel().astype(np.float32)
    col = (phi - np.float32(PHI_MIN)) / np.float32(PHI_MAX - PHI_MIN)
    col = col * np.float32(W_R - 1)
    col = np.clip(np.round(col), 0, W_R - 1).astype(np.int32)

    theta_low = np.arctan2(np.float32(Z_LOW), rho).astype(np.float32)
    row_low = (np.float32(THETA_MAX) - theta_low) / np.float32(
        THETA_MAX - THETA_MIN) * np.float32(H_R - 1)
    row_low = np.clip(np.round(row_low), 0, H_R - 1).astype(np.int32)

    order = np.argsort(col, kind="stable")
    counts = np.bincount(col, minlength=W_R)
    offs = np.zeros(W_R + 1, dtype=np.int64)
    np.cumsum(counts, out=offs[1:])

    blocks = []
    for blk in range(W_R // 128):
        kmax = int(counts[blk * 128:(blk + 1) * 128].max())
        K = ((kmax + 7) // 8) * 8
        idx_pad = np.zeros((K, 128), dtype=np.int32)
        valid = np.zeros((K, 128), dtype=bool)
        for j in range(128):
            c = blk * 128 + j
            n = counts[c]
            if n:
                idx_pad[:n, j] = order[offs[c]:offs[c + 1]]
                valid[:n, j] = True
        blocks.append((K, idx_pad, valid, rho[idx_pad], row_low[idx_pad]))
    return blocks


_BLOCKS = _static_layout()
_NBLK = len(_BLOCKS)
_SUMB = sum(b[0] * 128 for b in _BLOCKS)


_RG = 8


def _rv_body(feat_ref, s_ref, e_ref, o_ref):
    r0 = pl.program_id(1) * _RG
    s = s_ref[0]
    e = e_ref[0]
    neg = jnp.float32(-np.inf)
    for ch in range(32):
        f = feat_ref[0, ch]
        for j in range(_RG):
            r = r0 + j
            m = (s <= r) & (r <= e)
            v = jnp.where(m, f, neg)
            mx = jnp.max(v, axis=0)
            o_ref[0, ch, j, :] = jnp.where(mx == neg, jnp.float32(0.0), mx)


def _pallas_rv(featp, startp, endp, interpret=False):
    B, C, K = featp.shape[0], featp.shape[1], featp.shape[2]
    grid = (B, H_R // _RG)
    return pl.pallas_call(
        _rv_body,
        grid=grid,
        in_specs=[
            pl.BlockSpec((1, C, K, 128), lambda b, r: (b, 0, 0, 0)),
            pl.BlockSpec((1, K, 128), lambda b, r: (b, 0, 0)),
            pl.BlockSpec((1, K, 128), lambda b, r: (b, 0, 0)),
        ],
        out_specs=pl.BlockSpec((1, C, _RG, 128), lambda b, r: (b, 0, r, 0)),
        out_shape=jax.ShapeDtypeStruct((B, C, H_R, 128), jnp.float32),
        interpret=interpret,
    )(featp, startp, endp)


_NW = 32
_ROW = 128
_GCH = 128
_NB = 4
_SS = _GCH * _NB
_TOT = -(-(2 * _SUMB) // (_SS * _NW)) * _SS * _NW
_PER_W = _TOT // _NW
_NSS = _PER_W // _SS


def _gather_order():
    per_batch = np.concatenate(
        [b[1].reshape(-1) for b in _BLOCKS]).astype(np.int64)
    g = np.concatenate([per_batch, per_batch + _N])
    g = np.concatenate([g, np.zeros(_TOT - len(g), dtype=np.int64)])
    return g.astype(np.int32)


_GIDX = _gather_order()


def _sc_gather(src, idx):
    mesh = plsc.VectorSubcoreMesh(core_axis_name="c", subcore_axis_name="s")

    @functools.partial(
        pl.kernel,
        mesh=mesh,
        out_type=jax.ShapeDtypeStruct((_TOT, _ROW), jnp.float32),
        scratch_types=[
            pltpu.VMEM((_PER_W,), jnp.int32),
            pltpu.VMEM((_SS, _ROW), jnp.float32),
            pltpu.SemaphoreType.DMA,
            pltpu.SemaphoreType.DMA,
        ],
    )
    def gk(src_hbm, idx_hbm, out_hbm, idx_all, rows_v, sg, sw):
        wid = lax.axis_index("s") * 2 + lax.axis_index("c")
        base = wid * _PER_W

        def g_copies(s):
            for n in range(_NB):
                yield (src_hbm.at[idx_all.at[pl.ds(s * _SS + n * _GCH, _GCH)]],
                       rows_v.at[pl.ds(n * _GCH, _GCH)])

        def wb_args(s):
            return rows_v, out_hbm.at[pl.ds(base + s * _SS, _SS)]

        pltpu.sync_copy(idx_hbm.at[pl.ds(base, _PER_W)], idx_all)
        for a, b in g_copies(0):
            pltpu.async_copy(a, b, sg)

        def body(s, carry):
            for a, b in g_copies(s):
                pltpu.make_async_copy(a, b, sg).wait()
            pltpu.async_copy(*wb_args(s), sw)
            pltpu.make_async_copy(rows_v, out_hbm.at[pl.ds(base + s * _SS, _SS)], sw).wait()
            for a, b in g_copies(s + 1):
                pltpu.async_copy(a, b, sg)
            return carry

        lax.fori_loop(0, _NSS - 1, body, 0)

        s_last = _NSS - 1
        for a, b in g_copies(s_last):
            pltpu.make_async_copy(a, b, sg).wait()
        pltpu.async_copy(*wb_args(s_last), sw)
        pltpu.make_async_copy(rows_v, out_hbm.at[pl.ds(base + s_last * _SS, _SS)], sw).wait()

    return gk(src, idx)


def kernel(bev_feat, bev_z_bin):
    B, C = bev_feat.shape[0], bev_feat.shape[1]

    dz = (Z_MAX - Z_MIN) / Z_BINS
    z_hint = bev_z_bin[:, 0].astype(jnp.float32) * dz + (Z_MIN + dz / 2.0)
    bev_t = bev_feat.reshape(B, C, -1).transpose(0, 2, 1)
    src = jnp.concatenate(
        [bev_t, z_hint.reshape(B, _N, 1),
         jnp.zeros((B, _N, _ROW - C - 1), jnp.float32)], axis=2)
    src = src.reshape(B * _N, _ROW)

    rows = _sc_gather(src, jnp.asarray(_GIDX))

    outs = []
    off = 0
    for K, _idx, valid_np, rho_np, rl_np in _BLOCKS:
        L = K * 128
        rb = jnp.stack([rows[off:off + L], rows[_SUMB + off:_SUMB + off + L]])
        rb = rb.reshape(B, K, 128, _ROW)
        featb = rb[..., :C].transpose(0, 3, 1, 2)

        zb = rb[..., C]
        valid = jnp.asarray(valid_np)
        theta_high = jnp.arctan2(zb, jnp.asarray(rho_np)[None])
        row_high = (THETA_MAX - theta_high) / (THETA_MAX - THETA_MIN) * (H_R - 1)
        row_high = jnp.clip(jnp.round(row_high), 0, H_R - 1).astype(jnp.int32)
        rl = jnp.asarray(rl_np)[None]
        startb = jnp.where(valid[None], jnp.minimum(rl, row_high), jnp.int32(H_R))
        endb = jnp.where(valid[None], jnp.maximum(rl, row_high), jnp.int32(-1))
        outs.append(_pallas_rv(featb, startb, endb))
        off += L
    return jnp.concatenate(outs, axis=3)
